# baseline (speedup 1.0000x reference)
import functools

import jax
import jax.numpy as jnp
from jax import lax
from jax.experimental import pallas as pl
from jax.experimental.pallas import tpu as pltpu
from jax.experimental.pallas import tpu_sc as plsc

N_DET = 5000
N_GT = 1000
N_THR = 10
N_REC = 101

_B_PAD = 5120
_TAB_D = 128
_NC, _NS = 2, 16
_NW = _NC * _NS
_BPW = _B_PAD // _NW
_BD = 1000
_NB = N_DET // _BD


def _sc_gather(tab, idx):
    mesh = plsc.VectorSubcoreMesh(core_axis_name="c", subcore_axis_name="s")

    @functools.partial(
        pl.kernel, mesh=mesh,
        out_type=jax.ShapeDtypeStruct((_B_PAD, _TAB_D), jnp.float32),
        scratch_types=[
            pltpu.VMEM((_BPW,), jnp.int32),
            pltpu.VMEM((_BPW, _TAB_D), jnp.float32),
            pltpu.SemaphoreType.DMA,
        ],
    )
    def gk(tab_hbm, idx_hbm, out_hbm, idx_v, rows_v, sem):
        wid = lax.axis_index("s") * _NC + lax.axis_index("c")
        base = wid * _BPW
        pltpu.sync_copy(idx_hbm.at[pl.ds(base, _BPW)], idx_v)
        pltpu.async_copy(tab_hbm.at[idx_v], rows_v, sem).wait()
        pltpu.sync_copy(rows_v, out_hbm.at[pl.ds(base, _BPW)])

    return gk(tab, idx)


def _tc_body(det_ref, gt_ref, thr_ref, rthr_ref, iou_ref, map_ref,
             used_s, tp_s, ap_s):
    pid = pl.program_id(0)

    d = det_ref[...]
    g = gt_ref[...]
    p = None
    vd = None
    vg = None
    for a in range(3):
        dlo = jnp.min(d[:, 8 * a:8 * a + 8], axis=1, keepdims=True)
        dhi = jnp.max(d[:, 8 * a:8 * a + 8], axis=1, keepdims=True)
        glo = jnp.min(g[8 * a:8 * a + 8, :], axis=0, keepdims=True)
        ghi = jnp.max(g[8 * a:8 * a + 8, :], axis=0, keepdims=True)
        e = jnp.maximum(jnp.minimum(dhi, ghi) - jnp.maximum(dlo, glo), 0.0)
        p = e if p is None else p * e
        vd = (dhi - dlo) if vd is None else vd * (dhi - dlo)
        vg = (ghi - glo) if vg is None else vg * (ghi - glo)
    union = jnp.maximum(vd + vg - p, 1e-9)
    iou = p / union
    same = d[:, 24:25] == g[24:25, :]
    iou_ref[...] = jnp.where(same, iou, 0.0)

    @pl.when(pid == 0)
    def _init():
        used_s[...] = jnp.zeros((N_THR, N_GT), jnp.float32)
        tp_s[...] = jnp.zeros((N_THR, 128), jnp.float32)
        ap_s[...] = jnp.zeros((N_THR, 128), jnp.float32)

    thr = thr_ref[...]
    rthr = rthr_ref[...]
    iota = lax.broadcasted_iota(jnp.int32, (N_THR, N_GT), 1)

    def row_step(r, carry):
        used, tp, ap = carry
        row = iou_ref[pl.ds(r, 1), :]
        cand = jnp.broadcast_to(row, (N_THR, N_GT)) + used
        m = jnp.max(cand, axis=1, keepdims=True)
        ok = m >= thr
        first = jnp.min(jnp.where(cand == m, iota, N_GT), axis=1, keepdims=True)
        first = jnp.where(ok, first, -1)
        used = jnp.minimum(used, jnp.where(iota == first, -2.0, 0.0))
        tp = tp + ok.astype(jnp.float32)
        inv = 1.0 / (pid * _BD + r + 1).astype(jnp.float32)
        ap = jnp.maximum(ap,
                         jnp.where(tp * (1.0 / N_GT) >= rthr, tp * inv, 0.0))
        return used, tp, ap

    carry0 = (used_s[...], tp_s[:, 0:1], ap_s[...])
    used, tp, ap = lax.fori_loop(0, _BD, row_step, carry0)
    used_s[...] = used
    tp_s[...] = jnp.broadcast_to(tp, (N_THR, 128))
    ap_s[...] = ap

    @pl.when(pid == _NB - 1)
    def _fin():
        aps = jnp.sum(ap, axis=1, keepdims=True) * (1.0 / N_REC)
        map_ref[...] = jnp.sum(aps).reshape(1, 1) * (1.0 / N_THR)


def _tc_call(det, gtT, thr, rthr):
    return pl.pallas_call(
        _tc_body,
        grid=(_NB,),
        in_specs=[
            pl.BlockSpec((_BD, 32), lambda i: (i, 0)),
            pl.BlockSpec((32, N_GT), lambda i: (0, 0)),
            pl.BlockSpec((N_THR, 1), lambda i: (0, 0)),
            pl.BlockSpec((1, 128), lambda i: (0, 0)),
        ],
        out_specs=[
            pl.BlockSpec((_BD, N_GT), lambda i: (i, 0)),
            pl.BlockSpec((1, 1), lambda i: (0, 0)),
        ],
        out_shape=[
            jax.ShapeDtypeStruct((N_DET, N_GT), jnp.float32),
            jax.ShapeDtypeStruct((1, 1), jnp.float32),
        ],
        scratch_shapes=[
            pltpu.VMEM((N_THR, N_GT), jnp.float32),
            pltpu.VMEM((N_THR, 128), jnp.float32),
            pltpu.VMEM((N_THR, 128), jnp.float32),
        ],
        compiler_params=pltpu.CompilerParams(
            dimension_semantics=("arbitrary",)),
    )(det, gtT, thr, rthr)


def kernel(pred_boxes, pred_scores, pred_labels, gt_boxes, gt_labels):
    order = jnp.argsort(-pred_scores).astype(jnp.int32)
    idx = jnp.concatenate(
        [order, jnp.zeros((_B_PAD - N_DET,), jnp.int32)])

    def pack(boxes, labels, n):
        c = jnp.transpose(boxes, (0, 2, 1)).reshape(n, 24)
        return jnp.concatenate(
            [c, labels.astype(jnp.float32)[:, None],
             jnp.zeros((n, _TAB_D - 25), jnp.float32)], axis=1)

    tab = pack(pred_boxes, pred_labels, N_DET)
    det = _sc_gather(tab, idx)[:N_DET, :32]
    gtT = pack(gt_boxes, gt_labels, N_GT)[:, :32].T

    thr = jnp.linspace(0.05, 0.5, N_THR).astype(jnp.float32).reshape(N_THR, 1)
    rthr = jnp.concatenate(
        [jnp.linspace(0.0, 1.0, N_REC).astype(jnp.float32),
         jnp.full((128 - N_REC,), 2.0, jnp.float32)]).reshape(1, 128)

    ious, mapv = _tc_call(det, gtT, thr, rthr)
    return mapv[0, 0], ious


# speedup vs baseline: 1.3271x; 1.3271x over previous
import functools

import jax
import jax.numpy as jnp
from jax import lax
from jax.experimental import pallas as pl
from jax.experimental.pallas import tpu as pltpu
from jax.experimental.pallas import tpu_sc as plsc

N_DET = 5000
N_GT = 1000
N_THR = 10
N_REC = 101

_B_PAD = 5120
_TAB_D = 128
_NC, _NS = 2, 16
_NW = _NC * _NS
_BPW = _B_PAD // _NW
_BD = 1000
_NB = N_DET // _BD


def _sc_gather(tab, idx):
    mesh = plsc.VectorSubcoreMesh(core_axis_name="c", subcore_axis_name="s")

    @functools.partial(
        pl.kernel, mesh=mesh,
        out_type=jax.ShapeDtypeStruct((_B_PAD, _TAB_D), jnp.float32),
        scratch_types=[
            pltpu.VMEM((_BPW,), jnp.int32),
            pltpu.VMEM((_BPW, _TAB_D), jnp.float32),
            pltpu.SemaphoreType.DMA,
        ],
    )
    def gk(tab_hbm, idx_hbm, out_hbm, idx_v, rows_v, sem):
        wid = lax.axis_index("s") * _NC + lax.axis_index("c")
        base = wid * _BPW
        pltpu.sync_copy(idx_hbm.at[pl.ds(base, _BPW)], idx_v)
        pltpu.async_copy(tab_hbm.at[idx_v], rows_v, sem).wait()
        pltpu.sync_copy(rows_v, out_hbm.at[pl.ds(base, _BPW)])

    return gk(tab, idx)


def _tc_body(det_ref, gt_ref, thr_ref, rthr_ref, iou_ref, map_ref,
             used_s, tp_s, ap_s):
    pid = pl.program_id(0)

    d = det_ref[...]
    g = gt_ref[...]
    p = None
    vd = None
    vg = None
    for a in range(3):
        dlo = jnp.min(d[:, 8 * a:8 * a + 8], axis=1, keepdims=True)
        dhi = jnp.max(d[:, 8 * a:8 * a + 8], axis=1, keepdims=True)
        glo = jnp.min(g[8 * a:8 * a + 8, :], axis=0, keepdims=True)
        ghi = jnp.max(g[8 * a:8 * a + 8, :], axis=0, keepdims=True)
        e = jnp.maximum(jnp.minimum(dhi, ghi) - jnp.maximum(dlo, glo), 0.0)
        p = e if p is None else p * e
        vd = (dhi - dlo) if vd is None else vd * (dhi - dlo)
        vg = (ghi - glo) if vg is None else vg * (ghi - glo)
    union = jnp.maximum(vd + vg - p, 1e-9)
    iou = p / union
    same = d[:, 24:25] == g[24:25, :]
    iou_ref[...] = jnp.where(same, iou, 0.0)

    @pl.when(pid == 0)
    def _init():
        used_s[...] = jnp.zeros((N_THR, N_GT), jnp.float32)
        tp_s[...] = jnp.zeros((N_THR, 128), jnp.float32)
        ap_s[...] = jnp.zeros((N_THR, 128), jnp.float32)

    thr = thr_ref[...]
    rthr = rthr_ref[...]
    iota = lax.broadcasted_iota(jnp.int32, (N_THR, N_GT), 1)

    def chunk_step(c, carry):
        used, tp, ap = carry
        blk = iou_ref[pl.ds(c * 8, 8), :]
        for j in range(8):
            cand = jnp.where(used > 0.0, -1.0,
                             jnp.broadcast_to(blk[j:j + 1, :], (N_THR, N_GT)))
            m = jnp.max(cand, axis=1, keepdims=True)
            ok = m >= thr
            first = jnp.min(jnp.where(cand == m, iota, N_GT),
                            axis=1, keepdims=True)
            used = jnp.where((iota == first) & ok, 1.0, used)
            tp = tp + jnp.where(ok, 1.0, 0.0)
            inv = 1.0 / (pid * _BD + c * 8 + j + 1).astype(jnp.float32)
            ap = jnp.maximum(ap,
                             jnp.where(tp * (1.0 / N_GT) >= rthr, tp * inv, 0.0))
        return used, tp, ap

    carry0 = (used_s[...], tp_s[:, 0:1], ap_s[...])
    used, tp, ap = lax.fori_loop(0, _BD // 8, chunk_step, carry0)
    used_s[...] = used
    tp_s[...] = jnp.broadcast_to(tp, (N_THR, 128))
    ap_s[...] = ap

    @pl.when(pid == _NB - 1)
    def _fin():
        aps = jnp.sum(ap, axis=1, keepdims=True) * (1.0 / N_REC)
        map_ref[...] = jnp.sum(aps).reshape(1, 1) * (1.0 / N_THR)


def _tc_call(det, gtT, thr, rthr):
    return pl.pallas_call(
        _tc_body,
        grid=(_NB,),
        in_specs=[
            pl.BlockSpec((_BD, 32), lambda i: (i, 0)),
            pl.BlockSpec((32, N_GT), lambda i: (0, 0)),
            pl.BlockSpec((N_THR, 1), lambda i: (0, 0)),
            pl.BlockSpec((1, 128), lambda i: (0, 0)),
        ],
        out_specs=[
            pl.BlockSpec((_BD, N_GT), lambda i: (i, 0)),
            pl.BlockSpec((1, 1), lambda i: (0, 0)),
        ],
        out_shape=[
            jax.ShapeDtypeStruct((N_DET, N_GT), jnp.float32),
            jax.ShapeDtypeStruct((1, 1), jnp.float32),
        ],
        scratch_shapes=[
            pltpu.VMEM((N_THR, N_GT), jnp.float32),
            pltpu.VMEM((N_THR, 128), jnp.float32),
            pltpu.VMEM((N_THR, 128), jnp.float32),
        ],
        compiler_params=pltpu.CompilerParams(
            dimension_semantics=("arbitrary",)),
    )(det, gtT, thr, rthr)


def kernel(pred_boxes, pred_scores, pred_labels, gt_boxes, gt_labels):
    order = jnp.argsort(-pred_scores).astype(jnp.int32)
    idx = jnp.concatenate(
        [order, jnp.zeros((_B_PAD - N_DET,), jnp.int32)])

    def pack(boxes, labels, n):
        c = jnp.transpose(boxes, (0, 2, 1)).reshape(n, 24)
        return jnp.concatenate(
            [c, labels.astype(jnp.float32)[:, None],
             jnp.zeros((n, _TAB_D - 25), jnp.float32)], axis=1)

    tab = pack(pred_boxes, pred_labels, N_DET)
    det = _sc_gather(tab, idx)[:N_DET, :32]
    gtT = pack(gt_boxes, gt_labels, N_GT)[:, :32].T

    thr = jnp.linspace(0.05, 0.5, N_THR).astype(jnp.float32).reshape(N_THR, 1)
    rthr = jnp.concatenate(
        [jnp.linspace(0.0, 1.0, N_REC).astype(jnp.float32),
         jnp.full((128 - N_REC,), 2.0, jnp.float32)]).reshape(1, 128)

    ious, mapv = _tc_call(det, gtT, thr, rthr)
    return mapv[0, 0], ious
